# Initial kernel scaffold; baseline (speedup 1.0000x reference)
import jax
import jax.numpy as jnp
from jax.experimental import pallas as pl


def kernel(x, edge_index, batch, annotations, W1, b1, W2, b2, Wt1, bt1, Wt2, bt2, Wout, bout):
    raise NotImplementedError("write your pallas kernel here")


# speedup vs baseline: 27.8732x; 27.8732x over previous
import functools

import jax
import jax.numpy as jnp
from jax import lax
from jax.experimental import pallas as pl
from jax.experimental.pallas import tpu as pltpu
from jax.experimental.pallas import tpu_sc as plsc

_N = 10000
_E = 320000
_G = 64
_NC = 2
_NS = 16
_NW = _NC * _NS
_C = 80
_CH = (_E // _NW) // _C
_NB = 5
_NP = 10240
_RS = _NP // _NS

_f32 = jnp.float32


def _zero_fill(ref, rows, width):
    lanes = width // 16

    def body(k, _):
        i = k // lanes
        j = (k % lanes) * 16
        ref[i, pl.ds(j, 16)] = jnp.zeros((16,), _f32)
        return _

    lax.fori_loop(0, rows * lanes, body, None)


def _deg_body(dst3d, out, acc, dst_idx, ones_v, zbuf, sem):
    cid = lax.axis_index("c")
    sid = lax.axis_index("s")
    w = cid * _NS + sid

    def fill_ones(i, _):
        ones_v[i, :] = jnp.ones((16,), _f32)
        return _

    lax.fori_loop(0, _C, fill_ones, None)
    _zero_fill(zbuf, _RS, 16)
    pltpu.sync_copy(zbuf, acc.at[pl.ds(sid * _RS, _RS)])
    plsc.subcore_barrier()

    pltpu.sync_copy(dst3d.at[w], dst_idx)

    def super_step(s, _):
        base = s * _NB
        descs = []
        for b in range(_NB):
            descs.append(
                pltpu.async_copy(ones_v, acc.at[dst_idx.at[base + b]], sem,
                                 add=True))
        for d in descs:
            d.wait()
        return _

    lax.fori_loop(0, _CH // _NB, super_step, None)
    plsc.subcore_barrier()
    pltpu.sync_copy(acc.at[pl.ds(sid * _RS, _RS)],
                    out.at[cid, pl.ds(sid * _RS, _RS)])


def _deg_call(dst3d):
    mesh = plsc.VectorSubcoreMesh(core_axis_name="c", subcore_axis_name="s")
    return pl.kernel(
        _deg_body,
        out_type=jax.ShapeDtypeStruct((_NC, _NP, 16), _f32),
        mesh=mesh,
        compiler_params=pltpu.CompilerParams(use_tc_tiling_on_sc=False),
        scratch_types=[
            pltpu.VMEM_SHARED((_NP, 16), _f32),
            pltpu.VMEM((_CH, _C), jnp.int32),
            pltpu.VMEM((_C, 16), _f32),
            pltpu.VMEM((_RS, 16), _f32),
            pltpu.SemaphoreType.DMA,
        ],
    )(dst3d)


def _seg_body(table, src1d, dst3d, out, acc, src_idx, dst_idx, rows,
              gsem, ssem, *, width, nb):
    cid = lax.axis_index("c")
    sid = lax.axis_index("s")
    w = cid * _NS + sid

    _zero_fill(rows[0], _C, width)
    for t in range(_RS // _C):
        pltpu.sync_copy(rows[0], acc.at[pl.ds(sid * _RS + t * _C, _C)])
    plsc.subcore_barrier()

    pltpu.sync_copy(src1d.at[pl.ds(w * (_CH * _C), _CH * _C)], src_idx)
    pltpu.sync_copy(dst3d.at[w], dst_idx)

    def chunk_src(j):
        return src_idx.at[pl.ds(pl.multiple_of(j * _C, _C), _C)]

    def super_step(s, _):
        base = s * nb
        gd = []
        for b in range(nb):
            gd.append(
                pltpu.async_copy(table.at[chunk_src(base + b)], rows[b],
                                 gsem))
        for d in gd:
            d.wait()
        sd = []
        for b in range(nb):
            sd.append(
                pltpu.async_copy(rows[b], acc.at[dst_idx.at[base + b]], ssem,
                                 add=True))
        for d in sd:
            d.wait()
        return _

    supers = _CH // nb
    lax.fori_loop(0, supers, super_step, None)
    for j in range(supers * nb, _CH):
        pltpu.async_copy(table.at[chunk_src(j)], rows[0], gsem).wait()
        pltpu.async_copy(rows[0], acc.at[dst_idx.at[j]], ssem,
                         add=True).wait()
    plsc.subcore_barrier()
    pltpu.sync_copy(acc.at[pl.ds(sid * _RS, _RS)],
                    out.at[cid, pl.ds(sid * _RS, _RS)])


def _seg_call(table, src1d, dst3d, width):
    nb = 2 if width == 128 else 5
    mesh = plsc.VectorSubcoreMesh(core_axis_name="c", subcore_axis_name="s")
    return pl.kernel(
        functools.partial(_seg_body, width=width, nb=nb),
        out_type=jax.ShapeDtypeStruct((_NC, _NP, width), _f32),
        mesh=mesh,
        compiler_params=pltpu.CompilerParams(use_tc_tiling_on_sc=False),
        scratch_types=[
            pltpu.VMEM_SHARED((_NP, width), _f32),
            pltpu.VMEM((_CH * _C,), jnp.int32),
            pltpu.VMEM((_CH, _C), jnp.int32),
            [pltpu.VMEM((_C, width), _f32) for _ in range(nb)],
            pltpu.SemaphoreType.DMA,
            pltpu.SemaphoreType.DMA,
        ],
    )(table, src1d, dst3d)


def _dinv_from(dego_ref):
    deg = dego_ref[0, :_N] + dego_ref[1, :_N]
    d = deg[:, :1] + 1.0
    return lax.rsqrt(jnp.maximum(d, 1.0))


def _tc1_body(x_ref, w1_ref, dego_ref, hw1p_ref):
    dinv = _dinv_from(dego_ref)
    hw1p_ref[...] = jnp.dot(x_ref[...], w1_ref[...],
                            preferred_element_type=_f32) * dinv


def _tc2_body(s1_ref, hw1p_ref, dego_ref, w2_ref, b1_ref, hw2p_ref):
    dinv = _dinv_from(dego_ref)
    h1 = (dinv * (s1_ref[0, :_N] + s1_ref[1, :_N] + hw1p_ref[...])
          + b1_ref[...][None, :])
    h1 = jnp.maximum(h1, 0.0)
    hw2p_ref[...] = jnp.dot(h1, w2_ref[...],
                            preferred_element_type=_f32) * dinv


def _tc3_body(s2_ref, hw2p_ref, dego_ref, b2_ref, ann_ref, wt1c_ref,
              wt1a_ref, bt1_ref, wt2_ref, bt2_ref, wout_ref, bout_ref,
              batch_ref, xout_ref, concepts_ref, theta_ref):
    dinv = _dinv_from(dego_ref)
    h2 = (dinv * (s2_ref[0, :_N] + s2_ref[1, :_N] + hw2p_ref[...])
          + b2_ref[...][None, :])
    concepts = jax.nn.sigmoid(h2)
    pre = (jnp.dot(concepts, wt1c_ref[...], preferred_element_type=_f32)
           + jnp.dot(ann_ref[...], wt1a_ref[...], preferred_element_type=_f32)
           + bt1_ref[...][None, :])
    theta = (jnp.dot(jnp.tanh(pre), wt2_ref[...], preferred_element_type=_f32)
             + bt2_ref[...][None, :])
    contrib = theta * concepts
    gids = lax.broadcasted_iota(jnp.int32, (1, _G), 1)
    onehot = (batch_ref[...] == gids).astype(_f32)
    pooled = lax.dot_general(onehot, contrib,
                             (((0,), (0,)), ((), ())),
                             preferred_element_type=_f32)
    logits = jnp.dot(pooled, wout_ref[...],
                     preferred_element_type=_f32) + bout_ref[...][None, :]
    m = jnp.max(logits, axis=-1, keepdims=True)
    lse = jnp.log(jnp.sum(jnp.exp(logits - m), axis=-1, keepdims=True))
    xout_ref[...] = logits - m - lse
    concepts_ref[...] = concepts
    theta_ref[...] = theta


def kernel(x, edge_index, batch, annotations, W1, b1, W2, b2, Wt1, bt1,
           Wt2, bt2, Wout, bout):
    src3d = edge_index[0].reshape(_NW, _CH, _C)
    dst3d = edge_index[1].reshape(_NW, _CH, _C)

    dego = _deg_call(dst3d)

    hw1p = pl.pallas_call(
        _tc1_body,
        out_shape=jax.ShapeDtypeStruct((_N, 128), _f32),
    )(x, W1, dego)

    s1 = _seg_call(hw1p, edge_index[0], dst3d, 128)

    hw2p = pl.pallas_call(
        _tc2_body,
        out_shape=jax.ShapeDtypeStruct((_N, 64), _f32),
    )(s1, hw1p, dego, W2, b1)

    s2 = _seg_call(hw2p, edge_index[0], dst3d, 64)

    x_out, concepts, theta = pl.pallas_call(
        _tc3_body,
        out_shape=(
            jax.ShapeDtypeStruct((_G, 10), _f32),
            jax.ShapeDtypeStruct((_N, 64), _f32),
            jax.ShapeDtypeStruct((_N, 64), _f32),
        ),
    )(s2, hw2p, dego, b2, annotations, Wt1[:64], Wt1[64:], bt1, Wt2, bt2,
      Wout, bout, batch[:, None].astype(jnp.int32))

    return (x_out, concepts, theta, concepts)


# speedup vs baseline: 33.7008x; 1.2091x over previous
import functools

import jax
import jax.numpy as jnp
from jax import lax
from jax.experimental import pallas as pl
from jax.experimental.pallas import tpu as pltpu
from jax.experimental.pallas import tpu_sc as plsc

_N = 10000
_E = 320000
_G = 64
_NC = 2
_NS = 16
_NW = _NC * _NS
_C = 80
_CH = (_E // _NW) // _C
_NB = 5
_NP = 10240
_RS = _NP // _NS

_f32 = jnp.float32


def _zero_fill(ref, rows, width):
    lanes = width // 16

    def body(k, _):
        i = k // lanes
        j = (k % lanes) * 16
        ref[i, pl.ds(j, 16)] = jnp.zeros((16,), _f32)
        return _

    lax.fori_loop(0, rows * lanes, body, None)


def _deg_body(dst3d, out, acc, dst_idx, ones_v, zbuf, sem):
    cid = lax.axis_index("c")
    sid = lax.axis_index("s")
    w = cid * _NS + sid

    def fill_ones(i, _):
        ones_v[i, :] = jnp.ones((16,), _f32)
        return _

    lax.fori_loop(0, _C, fill_ones, None)
    _zero_fill(zbuf, _RS, 16)
    pltpu.sync_copy(zbuf, acc.at[pl.ds(sid * _RS, _RS)])
    plsc.subcore_barrier()

    pltpu.sync_copy(dst3d.at[w], dst_idx)

    def super_step(s, _):
        base = s * _NB
        descs = []
        for b in range(_NB):
            descs.append(
                pltpu.async_copy(ones_v, acc.at[dst_idx.at[base + b]], sem,
                                 add=True))
        for d in descs:
            d.wait()
        return _

    lax.fori_loop(0, _CH // _NB, super_step, None)
    plsc.subcore_barrier()
    pltpu.sync_copy(acc.at[pl.ds(sid * _RS, _RS)],
                    out.at[cid, pl.ds(sid * _RS, _RS)])


def _deg_call(dst3d):
    mesh = plsc.VectorSubcoreMesh(core_axis_name="c", subcore_axis_name="s")
    return pl.kernel(
        _deg_body,
        out_type=jax.ShapeDtypeStruct((_NC, _NP, 16), _f32),
        mesh=mesh,
        compiler_params=pltpu.CompilerParams(use_tc_tiling_on_sc=False),
        scratch_types=[
            pltpu.VMEM_SHARED((_NP, 16), _f32),
            pltpu.VMEM((_CH, _C), jnp.int32),
            pltpu.VMEM((_C, 16), _f32),
            pltpu.VMEM((_RS, 16), _f32),
            pltpu.SemaphoreType.DMA,
        ],
    )(dst3d)


def _seg_body(table, src1d, dst3d, out, acc, src_idx, dst_idx, rows,
              gsem, ssem, *, width, nb):
    cid = lax.axis_index("c")
    sid = lax.axis_index("s")
    w = cid * _NS + sid

    _zero_fill(rows[0], _C, width)
    for t in range(_RS // _C):
        pltpu.sync_copy(rows[0], acc.at[pl.ds(sid * _RS + t * _C, _C)])
    plsc.subcore_barrier()

    pltpu.sync_copy(src1d.at[pl.ds(w * (_CH * _C), _CH * _C)], src_idx)
    pltpu.sync_copy(dst3d.at[w], dst_idx)

    def chunk_src(j):
        return src_idx.at[pl.ds(pl.multiple_of(j * _C, _C), _C)]

    def gather(j, b):
        return pltpu.make_async_copy(table.at[chunk_src(j)], rows[b],
                                     gsem[b])

    def scatter(j, b):
        return pltpu.make_async_copy(rows[b], acc.at[dst_idx.at[j]],
                                     ssem[b])

    supers = _CH // nb
    for b in range(nb):
        gather(b, b).start()

    def super_step(s, _):
        base = s * nb
        for b in range(nb):
            gather(base + b, b).wait()
            scatter(base + b, b).start(add=True)

        @pl.when(s < supers - 1)
        def _refill():
            for b in range(nb):
                scatter(base + b, b).wait()
                gather(base + nb + b, b).start()

        return _

    lax.fori_loop(0, supers, super_step, None)
    for b in range(nb):
        scatter((supers - 1) * nb + b, b).wait()
    for j in range(supers * nb, _CH):
        gather(j, 0).start()
        gather(j, 0).wait()
        scatter(j, 0).start(add=True)
        scatter(j, 0).wait()
    plsc.subcore_barrier()
    pltpu.sync_copy(acc.at[pl.ds(sid * _RS, _RS)],
                    out.at[cid, pl.ds(sid * _RS, _RS)])


def _seg_call(table, src1d, dst3d, width):
    nb = 2 if width == 128 else 5
    mesh = plsc.VectorSubcoreMesh(core_axis_name="c", subcore_axis_name="s")
    return pl.kernel(
        functools.partial(_seg_body, width=width, nb=nb),
        out_type=jax.ShapeDtypeStruct((_NC, _NP, width), _f32),
        mesh=mesh,
        compiler_params=pltpu.CompilerParams(use_tc_tiling_on_sc=False),
        scratch_types=[
            pltpu.VMEM_SHARED((_NP, width), _f32),
            pltpu.VMEM((_CH * _C,), jnp.int32),
            pltpu.VMEM((_CH, _C), jnp.int32),
            [pltpu.VMEM((_C, width), _f32) for _ in range(nb)],
            [pltpu.SemaphoreType.DMA for _ in range(nb)],
            [pltpu.SemaphoreType.DMA for _ in range(nb)],
        ],
    )(table, src1d, dst3d)


def _dinv_from(dego_ref):
    deg = dego_ref[0, :_N] + dego_ref[1, :_N]
    d = deg[:, :1] + 1.0
    return lax.rsqrt(jnp.maximum(d, 1.0))


def _tc1_body(x_ref, w1_ref, dego_ref, hw1p_ref):
    dinv = _dinv_from(dego_ref)
    hw1p_ref[...] = jnp.dot(x_ref[...], w1_ref[...],
                            preferred_element_type=_f32) * dinv


def _tc2_body(s1_ref, hw1p_ref, dego_ref, w2_ref, b1_ref, hw2p_ref):
    dinv = _dinv_from(dego_ref)
    h1 = (dinv * (s1_ref[0, :_N] + s1_ref[1, :_N] + hw1p_ref[...])
          + b1_ref[...][None, :])
    h1 = jnp.maximum(h1, 0.0)
    hw2p_ref[...] = jnp.dot(h1, w2_ref[...],
                            preferred_element_type=_f32) * dinv


def _tc3_body(s2_ref, hw2p_ref, dego_ref, b2_ref, ann_ref, wt1c_ref,
              wt1a_ref, bt1_ref, wt2_ref, bt2_ref, wout_ref, bout_ref,
              batch_ref, xout_ref, concepts_ref, theta_ref):
    dinv = _dinv_from(dego_ref)
    h2 = (dinv * (s2_ref[0, :_N] + s2_ref[1, :_N] + hw2p_ref[...])
          + b2_ref[...][None, :])
    concepts = jax.nn.sigmoid(h2)
    pre = (jnp.dot(concepts, wt1c_ref[...], preferred_element_type=_f32)
           + jnp.dot(ann_ref[...], wt1a_ref[...], preferred_element_type=_f32)
           + bt1_ref[...][None, :])
    theta = (jnp.dot(jnp.tanh(pre), wt2_ref[...], preferred_element_type=_f32)
             + bt2_ref[...][None, :])
    contrib = theta * concepts
    gids = lax.broadcasted_iota(jnp.int32, (1, _G), 1)
    onehot = (batch_ref[...] == gids).astype(_f32)
    pooled = lax.dot_general(onehot, contrib,
                             (((0,), (0,)), ((), ())),
                             preferred_element_type=_f32)
    logits = jnp.dot(pooled, wout_ref[...],
                     preferred_element_type=_f32) + bout_ref[...][None, :]
    m = jnp.max(logits, axis=-1, keepdims=True)
    lse = jnp.log(jnp.sum(jnp.exp(logits - m), axis=-1, keepdims=True))
    xout_ref[...] = logits - m - lse
    concepts_ref[...] = concepts
    theta_ref[...] = theta


def kernel(x, edge_index, batch, annotations, W1, b1, W2, b2, Wt1, bt1,
           Wt2, bt2, Wout, bout):
    src3d = edge_index[0].reshape(_NW, _CH, _C)
    dst3d = edge_index[1].reshape(_NW, _CH, _C)

    dego = _deg_call(dst3d)

    hw1p = pl.pallas_call(
        _tc1_body,
        out_shape=jax.ShapeDtypeStruct((_N, 128), _f32),
    )(x, W1, dego)

    s1 = _seg_call(hw1p, edge_index[0], dst3d, 128)

    hw2p = pl.pallas_call(
        _tc2_body,
        out_shape=jax.ShapeDtypeStruct((_N, 64), _f32),
    )(s1, hw1p, dego, W2, b1)

    s2 = _seg_call(hw2p, edge_index[0], dst3d, 64)

    x_out, concepts, theta = pl.pallas_call(
        _tc3_body,
        out_shape=(
            jax.ShapeDtypeStruct((_G, 10), _f32),
            jax.ShapeDtypeStruct((_N, 64), _f32),
            jax.ShapeDtypeStruct((_N, 64), _f32),
        ),
    )(s2, hw2p, dego, b2, annotations, Wt1[:64], Wt1[64:], bt1, Wt2, bt2,
      Wout, bout, batch[:, None].astype(jnp.int32))

    return (x_out, concepts, theta, concepts)


# speedup vs baseline: 34.2826x; 1.0173x over previous
import functools

import jax
import jax.numpy as jnp
from jax import lax
from jax.experimental import pallas as pl
from jax.experimental.pallas import tpu as pltpu
from jax.experimental.pallas import tpu_sc as plsc

_N = 10000
_E = 320000
_G = 64
_NC = 2
_NS = 16
_NW = _NC * _NS
_C = 80
_CH = (_E // _NW) // _C
_NB = 5
_NP = 10240
_RS = _NP // _NS

_f32 = jnp.float32


def _zero_fill(ref, rows, width):
    lanes = width // 16

    def body(k, _):
        i = k // lanes
        j = (k % lanes) * 16
        ref[i, pl.ds(j, 16)] = jnp.zeros((16,), _f32)
        return _

    lax.fori_loop(0, rows * lanes, body, None)


def _deg_body(dst3d, out, acc, dst_idx, ones_v, zbuf, sem):
    cid = lax.axis_index("c")
    sid = lax.axis_index("s")
    w = cid * _NS + sid

    def fill_ones(i, _):
        ones_v[i, :] = jnp.ones((16,), _f32)
        return _

    lax.fori_loop(0, _C, fill_ones, None)
    _zero_fill(zbuf, _RS, 16)
    pltpu.sync_copy(zbuf, acc.at[pl.ds(sid * _RS, _RS)])
    plsc.subcore_barrier()

    pltpu.sync_copy(dst3d.at[w], dst_idx)

    def super_step(s, _):
        base = s * _NB
        descs = []
        for b in range(_NB):
            descs.append(
                pltpu.async_copy(ones_v, acc.at[dst_idx.at[base + b]], sem,
                                 add=True))
        for d in descs:
            d.wait()
        return _

    lax.fori_loop(0, _CH // _NB, super_step, None)
    plsc.subcore_barrier()
    pltpu.sync_copy(acc.at[pl.ds(sid * _RS, _RS)],
                    out.at[cid, pl.ds(sid * _RS, _RS)])


def _deg_call(dst3d):
    mesh = plsc.VectorSubcoreMesh(core_axis_name="c", subcore_axis_name="s")
    return pl.kernel(
        _deg_body,
        out_type=jax.ShapeDtypeStruct((_NC, _NP, 16), _f32),
        mesh=mesh,
        compiler_params=pltpu.CompilerParams(use_tc_tiling_on_sc=False),
        scratch_types=[
            pltpu.VMEM_SHARED((_NP, 16), _f32),
            pltpu.VMEM((_CH, _C), jnp.int32),
            pltpu.VMEM((_C, 16), _f32),
            pltpu.VMEM((_RS, 16), _f32),
            pltpu.SemaphoreType.DMA,
        ],
    )(dst3d)


def _seg_body(table, src1d, dst1d, out, acc, src_idx, dst_idx, rows,
              gsem, ssem, *, width, nb, cw, chunks):
    cid = lax.axis_index("c")
    sid = lax.axis_index("s")
    w = cid * _NS + sid
    ew = cw * chunks

    _zero_fill(rows[0], cw, width)
    for t in range(_RS // cw):
        pltpu.sync_copy(rows[0], acc.at[pl.ds(sid * _RS + t * cw, cw)])
    plsc.subcore_barrier()

    pltpu.sync_copy(src1d.at[pl.ds(w * ew, ew)], src_idx)
    pltpu.sync_copy(dst1d.at[pl.ds(w * ew, ew)], dst_idx)

    def chunk(ref, j):
        return ref.at[pl.ds(pl.multiple_of(j * cw, cw), cw)]

    def gather(j, b):
        return pltpu.make_async_copy(table.at[chunk(src_idx, j)], rows[b],
                                     gsem[b])

    def scatter(j, b):
        return pltpu.make_async_copy(rows[b], acc.at[chunk(dst_idx, j)],
                                     ssem[b])

    supers = chunks // nb
    for b in range(nb):
        gather(b, b).start()

    def super_step(s, _):
        base = s * nb
        for b in range(nb):
            gather(base + b, b).wait()
            scatter(base + b, b).start(add=True)

        @pl.when(s < supers - 1)
        def _refill():
            for b in range(nb):
                scatter(base + b, b).wait()
                gather(base + nb + b, b).start()

        return _

    lax.fori_loop(0, supers, super_step, None)
    for b in range(nb):
        scatter((supers - 1) * nb + b, b).wait()
    for j in range(supers * nb, chunks):
        gather(j, 0).start()
        gather(j, 0).wait()
        scatter(j, 0).start(add=True)
        scatter(j, 0).wait()
    plsc.subcore_barrier()
    pltpu.sync_copy(acc.at[pl.ds(sid * _RS, _RS)],
                    out.at[cid, pl.ds(sid * _RS, _RS)])


def _seg_call(table, src1d, dst1d, width):
    nb = 5 if width == 128 else 8
    cw = 40
    chunks = (_E // _NW) // cw
    mesh = plsc.VectorSubcoreMesh(core_axis_name="c", subcore_axis_name="s")
    return pl.kernel(
        functools.partial(_seg_body, width=width, nb=nb, cw=cw,
                          chunks=chunks),
        out_type=jax.ShapeDtypeStruct((_NC, _NP, width), _f32),
        mesh=mesh,
        compiler_params=pltpu.CompilerParams(use_tc_tiling_on_sc=False),
        scratch_types=[
            pltpu.VMEM_SHARED((_NP, width), _f32),
            pltpu.VMEM((cw * chunks,), jnp.int32),
            pltpu.VMEM((cw * chunks,), jnp.int32),
            [pltpu.VMEM((cw, width), _f32) for _ in range(nb)],
            [pltpu.SemaphoreType.DMA for _ in range(nb)],
            [pltpu.SemaphoreType.DMA for _ in range(nb)],
        ],
    )(table, src1d, dst1d)


def _dinv_from(dego_ref):
    deg = dego_ref[0, :_N] + dego_ref[1, :_N]
    d = deg[:, :1] + 1.0
    return lax.rsqrt(jnp.maximum(d, 1.0))


def _tc1_body(x_ref, w1_ref, dego_ref, hw1p_ref):
    dinv = _dinv_from(dego_ref)
    hw1p_ref[...] = jnp.dot(x_ref[...], w1_ref[...],
                            preferred_element_type=_f32) * dinv


def _tc2_body(s1_ref, hw1p_ref, dego_ref, w2_ref, b1_ref, hw2p_ref):
    dinv = _dinv_from(dego_ref)
    h1 = (dinv * (s1_ref[0, :_N] + s1_ref[1, :_N] + hw1p_ref[...])
          + b1_ref[...][None, :])
    h1 = jnp.maximum(h1, 0.0)
    hw2p_ref[...] = jnp.dot(h1, w2_ref[...],
                            preferred_element_type=_f32) * dinv


def _tc3_body(s2_ref, hw2p_ref, dego_ref, b2_ref, ann_ref, wt1c_ref,
              wt1a_ref, bt1_ref, wt2_ref, bt2_ref, wout_ref, bout_ref,
              batch_ref, xout_ref, concepts_ref, theta_ref):
    dinv = _dinv_from(dego_ref)
    h2 = (dinv * (s2_ref[0, :_N] + s2_ref[1, :_N] + hw2p_ref[...])
          + b2_ref[...][None, :])
    concepts = jax.nn.sigmoid(h2)
    pre = (jnp.dot(concepts, wt1c_ref[...], preferred_element_type=_f32)
           + jnp.dot(ann_ref[...], wt1a_ref[...], preferred_element_type=_f32)
           + bt1_ref[...][None, :])
    theta = (jnp.dot(jnp.tanh(pre), wt2_ref[...], preferred_element_type=_f32)
             + bt2_ref[...][None, :])
    contrib = theta * concepts
    gids = lax.broadcasted_iota(jnp.int32, (1, _G), 1)
    onehot = (batch_ref[...] == gids).astype(_f32)
    pooled = lax.dot_general(onehot, contrib,
                             (((0,), (0,)), ((), ())),
                             preferred_element_type=_f32)
    logits = jnp.dot(pooled, wout_ref[...],
                     preferred_element_type=_f32) + bout_ref[...][None, :]
    m = jnp.max(logits, axis=-1, keepdims=True)
    lse = jnp.log(jnp.sum(jnp.exp(logits - m), axis=-1, keepdims=True))
    xout_ref[...] = logits - m - lse
    concepts_ref[...] = concepts
    theta_ref[...] = theta


def kernel(x, edge_index, batch, annotations, W1, b1, W2, b2, Wt1, bt1,
           Wt2, bt2, Wout, bout):
    src3d = edge_index[0].reshape(_NW, _CH, _C)
    dst3d = edge_index[1].reshape(_NW, _CH, _C)

    dego = _deg_call(dst3d)

    hw1p = pl.pallas_call(
        _tc1_body,
        out_shape=jax.ShapeDtypeStruct((_N, 128), _f32),
    )(x, W1, dego)

    s1 = _seg_call(hw1p, edge_index[0], edge_index[1], 128)

    hw2p = pl.pallas_call(
        _tc2_body,
        out_shape=jax.ShapeDtypeStruct((_N, 64), _f32),
    )(s1, hw1p, dego, W2, b1)

    s2 = _seg_call(hw2p, edge_index[0], edge_index[1], 64)

    x_out, concepts, theta = pl.pallas_call(
        _tc3_body,
        out_shape=(
            jax.ShapeDtypeStruct((_G, 10), _f32),
            jax.ShapeDtypeStruct((_N, 64), _f32),
            jax.ShapeDtypeStruct((_N, 64), _f32),
        ),
    )(s2, hw2p, dego, b2, annotations, Wt1[:64], Wt1[64:], bt1, Wt2, bt2,
      Wout, bout, batch[:, None].astype(jnp.int32))

    return (x_out, concepts, theta, concepts)


# speedup vs baseline: 35.1775x; 1.0261x over previous
import functools

import jax
import jax.numpy as jnp
from jax import lax
from jax.experimental import pallas as pl
from jax.experimental.pallas import tpu as pltpu
from jax.experimental.pallas import tpu_sc as plsc

_N = 10000
_E = 320000
_G = 64
_NC = 2
_NS = 16
_NW = _NC * _NS
_C = 80
_CH = (_E // _NW) // _C
_NB = 5
_NP = 10240
_RS = _NP // _NS

_f32 = jnp.float32


def _zero_fill(ref, rows, width):
    lanes = width // 16

    def body(k, _):
        i = k // lanes
        j = (k % lanes) * 16
        ref[i, pl.ds(j, 16)] = jnp.zeros((16,), _f32)
        return _

    lax.fori_loop(0, rows * lanes, body, None)


def _deg_body(edge, out, acc, dst_idx, ones_v, zbuf, sem):
    cid = lax.axis_index("c")
    sid = lax.axis_index("s")
    w = cid * _NS + sid

    def fill_ones(i, _):
        ones_v[i, :] = jnp.ones((16,), _f32)
        return _

    lax.fori_loop(0, _C, fill_ones, None)
    _zero_fill(zbuf, _RS, 16)
    pltpu.sync_copy(zbuf, acc.at[pl.ds(sid * _RS, _RS)])
    plsc.subcore_barrier()

    pltpu.sync_copy(edge.at[1, pl.ds(w * (_CH * _C), _CH * _C)], dst_idx)

    def super_step(s, _):
        base = s * _NB
        descs = []
        for b in range(_NB):
            idx = dst_idx.at[pl.ds(pl.multiple_of((base + b) * _C, _C), _C)]
            descs.append(
                pltpu.async_copy(ones_v, acc.at[idx], sem, add=True))
        for d in descs:
            d.wait()
        return _

    lax.fori_loop(0, _CH // _NB, super_step, None)
    plsc.subcore_barrier()
    pltpu.sync_copy(acc.at[pl.ds(sid * _RS, _RS)],
                    out.at[cid, pl.ds(sid * _RS, _RS)])


def _deg_call(edge_index):
    mesh = plsc.VectorSubcoreMesh(core_axis_name="c", subcore_axis_name="s")
    return pl.kernel(
        _deg_body,
        out_type=jax.ShapeDtypeStruct((_NC, _NP, 16), _f32),
        mesh=mesh,
        compiler_params=pltpu.CompilerParams(use_tc_tiling_on_sc=False),
        scratch_types=[
            pltpu.VMEM_SHARED((_NP, 16), _f32),
            pltpu.VMEM((_CH * _C,), jnp.int32),
            pltpu.VMEM((_C, 16), _f32),
            pltpu.VMEM((_RS, 16), _f32),
            pltpu.SemaphoreType.DMA,
        ],
    )(edge_index)


def _seg_body(table, edge, out, acc, src_idx, dst_idx, rows,
              gsem, ssem, *, width, nb, cw, chunks):
    cid = lax.axis_index("c")
    sid = lax.axis_index("s")
    w = cid * _NS + sid
    ew = cw * chunks

    _zero_fill(rows[0], cw, width)
    for t in range(_RS // cw):
        pltpu.sync_copy(rows[0], acc.at[pl.ds(sid * _RS + t * cw, cw)])
    plsc.subcore_barrier()

    pltpu.sync_copy(edge.at[0, pl.ds(w * ew, ew)], src_idx)
    pltpu.sync_copy(edge.at[1, pl.ds(w * ew, ew)], dst_idx)

    def chunk(ref, j):
        return ref.at[pl.ds(pl.multiple_of(j * cw, cw), cw)]

    def gather(j, b):
        return pltpu.make_async_copy(table.at[chunk(src_idx, j)], rows[b],
                                     gsem[b])

    def scatter(j, b):
        return pltpu.make_async_copy(rows[b], acc.at[chunk(dst_idx, j)],
                                     ssem[b])

    supers = chunks // nb
    for b in range(nb):
        gather(b, b).start()

    def super_step(s, _):
        base = s * nb
        for b in range(nb):
            gather(base + b, b).wait()
            scatter(base + b, b).start(add=True)

        @pl.when(s < supers - 1)
        def _refill():
            for b in range(nb):
                scatter(base + b, b).wait()
                gather(base + nb + b, b).start()

        return _

    lax.fori_loop(0, supers, super_step, None)
    for b in range(nb):
        scatter((supers - 1) * nb + b, b).wait()
    for j in range(supers * nb, chunks):
        gather(j, 0).start()
        gather(j, 0).wait()
        scatter(j, 0).start(add=True)
        scatter(j, 0).wait()
    plsc.subcore_barrier()
    pltpu.sync_copy(acc.at[pl.ds(sid * _RS, _RS)],
                    out.at[cid, pl.ds(sid * _RS, _RS)])


def _seg_call(table, edge_index, width):
    nb = 5 if width == 128 else 8
    cw = 40
    chunks = (_E // _NW) // cw
    mesh = plsc.VectorSubcoreMesh(core_axis_name="c", subcore_axis_name="s")
    return pl.kernel(
        functools.partial(_seg_body, width=width, nb=nb, cw=cw,
                          chunks=chunks),
        out_type=jax.ShapeDtypeStruct((_NC, _NP, width), _f32),
        mesh=mesh,
        compiler_params=pltpu.CompilerParams(use_tc_tiling_on_sc=False),
        scratch_types=[
            pltpu.VMEM_SHARED((_NP, width), _f32),
            pltpu.VMEM((cw * chunks,), jnp.int32),
            pltpu.VMEM((cw * chunks,), jnp.int32),
            [pltpu.VMEM((cw, width), _f32) for _ in range(nb)],
            [pltpu.SemaphoreType.DMA for _ in range(nb)],
            [pltpu.SemaphoreType.DMA for _ in range(nb)],
        ],
    )(table, edge_index)


def _dinv_from(dego_ref):
    deg = dego_ref[0, :_N] + dego_ref[1, :_N]
    d = deg[:, :1] + 1.0
    return lax.rsqrt(jnp.maximum(d, 1.0))


def _tc1_body(x_ref, w1_ref, dego_ref, hw1p_ref, dinv_ref):
    dinv = _dinv_from(dego_ref)
    dinv_ref[...] = jnp.broadcast_to(dinv, (_N, 128))
    hw1p_ref[...] = jnp.dot(x_ref[...], w1_ref[...],
                            preferred_element_type=_f32) * dinv


def _tc2_body(s1_ref, hw1p_ref, dinv_ref, w2_ref, b1_ref, hw2p_ref):
    dinv = dinv_ref[...]
    h1 = (dinv * (s1_ref[0, :_N] + s1_ref[1, :_N] + hw1p_ref[...])
          + b1_ref[...][None, :])
    h1 = jnp.maximum(h1, 0.0)
    hw2p_ref[...] = jnp.dot(h1, w2_ref[...],
                            preferred_element_type=_f32) * dinv[:, :64]


def _tc3_body(s2_ref, hw2p_ref, dinv_ref, b2_ref, ann_ref, wt1c_ref,
              wt1a_ref, bt1_ref, wt2_ref, bt2_ref, wout_ref, bout_ref,
              batch_ref, xout_ref, concepts_ref, theta_ref, concepts2_ref):
    dinv = dinv_ref[...][:, :64]
    h2 = (dinv * (s2_ref[0, :_N] + s2_ref[1, :_N] + hw2p_ref[...])
          + b2_ref[...][None, :])
    concepts = jax.nn.sigmoid(h2)
    pre = (jnp.dot(concepts, wt1c_ref[...], preferred_element_type=_f32)
           + jnp.dot(ann_ref[...], wt1a_ref[...], preferred_element_type=_f32)
           + bt1_ref[...][None, :])
    theta = (jnp.dot(jnp.tanh(pre), wt2_ref[...], preferred_element_type=_f32)
             + bt2_ref[...][None, :])
    contrib = theta * concepts
    gids = lax.broadcasted_iota(jnp.int32, (1, _G), 1)
    onehot = (batch_ref[...] == gids).astype(_f32)
    pooled = lax.dot_general(onehot, contrib,
                             (((0,), (0,)), ((), ())),
                             preferred_element_type=_f32)
    logits = jnp.dot(pooled, wout_ref[...],
                     preferred_element_type=_f32) + bout_ref[...][None, :]
    m = jnp.max(logits, axis=-1, keepdims=True)
    lse = jnp.log(jnp.sum(jnp.exp(logits - m), axis=-1, keepdims=True))
    xout_ref[...] = logits - m - lse
    concepts_ref[...] = concepts
    theta_ref[...] = theta
    concepts2_ref[...] = concepts


def kernel(x, edge_index, batch, annotations, W1, b1, W2, b2, Wt1, bt1,
           Wt2, bt2, Wout, bout):
    dego = _deg_call(edge_index)

    hw1p, dinv = pl.pallas_call(
        _tc1_body,
        out_shape=(
            jax.ShapeDtypeStruct((_N, 128), _f32),
            jax.ShapeDtypeStruct((_N, 128), _f32),
        ),
    )(x, W1, dego)

    s1 = _seg_call(hw1p, edge_index, 128)

    hw2p = pl.pallas_call(
        _tc2_body,
        out_shape=jax.ShapeDtypeStruct((_N, 64), _f32),
    )(s1, hw1p, dinv, W2, b1)

    s2 = _seg_call(hw2p, edge_index, 64)

    x_out, concepts, theta, concepts2 = pl.pallas_call(
        _tc3_body,
        out_shape=(
            jax.ShapeDtypeStruct((_G, 10), _f32),
            jax.ShapeDtypeStruct((_N, 64), _f32),
            jax.ShapeDtypeStruct((_N, 64), _f32),
            jax.ShapeDtypeStruct((_N, 64), _f32),
        ),
    )(s2, hw2p, dinv, b2, annotations, Wt1[:64], Wt1[64:], bt1, Wt2, bt2,
      Wout, bout, batch[:, None].astype(jnp.int32))

    return (x_out, concepts, theta, concepts2)


# speedup vs baseline: 35.4120x; 1.0067x over previous
import functools

import jax
import jax.numpy as jnp
from jax import lax
from jax.experimental import pallas as pl
from jax.experimental.pallas import tpu as pltpu
from jax.experimental.pallas import tpu_sc as plsc

_N = 10000
_E = 320000
_G = 64
_NC = 2
_NS = 16
_NW = _NC * _NS
_C = 80
_CH = (_E // _NW) // _C
_NB = 5
_NP = 10240
_RS = _NP // _NS

_f32 = jnp.float32


def _zero_fill(ref, rows, width):
    lanes = width // 16

    def body(k, _):
        i = k // lanes
        j = (k % lanes) * 16
        ref[i, pl.ds(j, 16)] = jnp.zeros((16,), _f32)
        return _

    lax.fori_loop(0, rows * lanes, body, None)


def _deg_body(edge, out, acc, dst_idx, ones_v, zbuf, sem):
    cid = lax.axis_index("c")
    sid = lax.axis_index("s")
    w = cid * _NS + sid

    def fill_ones(i, _):
        ones_v[i, :] = jnp.ones((16,), _f32)
        return _

    lax.fori_loop(0, _C, fill_ones, None)
    _zero_fill(zbuf, _RS, 16)
    pltpu.sync_copy(zbuf, acc.at[pl.ds(sid * _RS, _RS)])
    plsc.subcore_barrier()

    pltpu.sync_copy(edge.at[1, pl.ds(w * (_CH * _C), _CH * _C)], dst_idx)

    def super_step(s, _):
        base = s * _NB
        descs = []
        for b in range(_NB):
            idx = dst_idx.at[pl.ds(pl.multiple_of((base + b) * _C, _C), _C)]
            descs.append(
                pltpu.async_copy(ones_v, acc.at[idx], sem, add=True))
        for d in descs:
            d.wait()
        return _

    lax.fori_loop(0, _CH // _NB, super_step, None)
    plsc.subcore_barrier()
    pltpu.sync_copy(acc.at[pl.ds(sid * _RS, _RS)],
                    out.at[cid, pl.ds(sid * _RS, _RS)])


def _deg_call(edge_index):
    mesh = plsc.VectorSubcoreMesh(core_axis_name="c", subcore_axis_name="s")
    return pl.kernel(
        _deg_body,
        out_type=jax.ShapeDtypeStruct((_NC, _NP, 16), _f32),
        mesh=mesh,
        compiler_params=pltpu.CompilerParams(use_tc_tiling_on_sc=False),
        scratch_types=[
            pltpu.VMEM_SHARED((_NP, 16), _f32),
            pltpu.VMEM((_CH * _C,), jnp.int32),
            pltpu.VMEM((_C, 16), _f32),
            pltpu.VMEM((_RS, 16), _f32),
            pltpu.SemaphoreType.DMA,
        ],
    )(edge_index)


def _seg_body(table, edge, out, acc, src_idx, dst_idx, rows,
              gsem, ssem, *, width, nb, cw, chunks):
    cid = lax.axis_index("c")
    sid = lax.axis_index("s")
    w = cid * _NS + sid
    ew = cw * chunks

    _zero_fill(rows[0], cw, width)
    for t in range(_RS // cw):
        pltpu.sync_copy(rows[0], acc.at[pl.ds(sid * _RS + t * cw, cw)])
    plsc.subcore_barrier()

    pltpu.sync_copy(edge.at[0, pl.ds(w * ew, ew)], src_idx)
    pltpu.sync_copy(edge.at[1, pl.ds(w * ew, ew)], dst_idx)

    def chunk(ref, j):
        return ref.at[pl.ds(pl.multiple_of(j * cw, cw), cw)]

    def gather(j, b):
        return pltpu.make_async_copy(table.at[chunk(src_idx, j)], rows[b],
                                     gsem[b])

    def scatter(j, b):
        return pltpu.make_async_copy(rows[b], acc.at[chunk(dst_idx, j)],
                                     ssem[b])

    supers = chunks // nb
    for b in range(nb):
        gather(b, b).start()

    def super_step(s, _):
        base = s * nb
        for b in range(nb):
            gather(base + b, b).wait()
            scatter(base + b, b).start(add=True)

        @pl.when(s < supers - 1)
        def _refill():
            for b in range(nb):
                scatter(base + b, b).wait()
                gather(base + nb + b, b).start()

        return _

    lax.fori_loop(0, supers, super_step, None)
    for b in range(nb):
        scatter((supers - 1) * nb + b, b).wait()
    for j in range(supers * nb, chunks):
        gather(j, 0).start()
        gather(j, 0).wait()
        scatter(j, 0).start(add=True)
        scatter(j, 0).wait()
    plsc.subcore_barrier()
    pltpu.sync_copy(acc.at[pl.ds(sid * _RS, _RS)],
                    out.at[cid, pl.ds(sid * _RS, _RS)])


def _seg_call(table, edge_index, width):
    nb = 5 if width == 128 else 8
    cw = 40
    chunks = (_E // _NW) // cw
    mesh = plsc.VectorSubcoreMesh(core_axis_name="c", subcore_axis_name="s")
    return pl.kernel(
        functools.partial(_seg_body, width=width, nb=nb, cw=cw,
                          chunks=chunks),
        out_type=jax.ShapeDtypeStruct((_NC, _NP, width), _f32),
        mesh=mesh,
        compiler_params=pltpu.CompilerParams(use_tc_tiling_on_sc=False),
        scratch_types=[
            pltpu.VMEM_SHARED((_NP, width), _f32),
            pltpu.VMEM((cw * chunks,), jnp.int32),
            pltpu.VMEM((cw * chunks,), jnp.int32),
            [pltpu.VMEM((cw, width), _f32) for _ in range(nb)],
            [pltpu.SemaphoreType.DMA for _ in range(nb)],
            [pltpu.SemaphoreType.DMA for _ in range(nb)],
        ],
    )(table, edge_index)


def _dinv_from(dego_ref):
    deg = dego_ref[0, :_N] + dego_ref[1, :_N]
    d = deg[:, :1] + 1.0
    return lax.rsqrt(jnp.maximum(d, 1.0))


def _tc1_body(x_ref, w1_ref, dego_ref, hw1p_ref, dinv_ref):
    dinv = _dinv_from(dego_ref)
    dinv_ref[...] = jnp.broadcast_to(dinv, (_N, 128))
    hw1p_ref[...] = jnp.dot(x_ref[...], w1_ref[...],
                            preferred_element_type=_f32) * dinv


def _tc2_body(s1_ref, hw1p_ref, dinv_ref, w2_ref, b1_ref, hw2p_ref):
    dinv = dinv_ref[...]
    h1 = (dinv * (s1_ref[0, :_N] + s1_ref[1, :_N] + hw1p_ref[...])
          + b1_ref[...][None, :])
    h1 = jnp.maximum(h1, 0.0)
    hw2p_ref[...] = jnp.dot(h1, w2_ref[...],
                            preferred_element_type=_f32) * dinv[:, :64]


def _pre_a_body(ann_ref, wt1a_ref, bt1_ref, pre_a_ref):
    pre_a_ref[...] = (jnp.dot(ann_ref[...], wt1a_ref[...],
                              preferred_element_type=_f32)
                      + bt1_ref[...][None, :])


def _tc3_body(s2_ref, hw2p_ref, dinv_ref, b2_ref, pre_a_ref, wt1c_ref,
              wt2_ref, bt2_ref, wout_ref, bout_ref,
              batch_ref, xout_ref, concepts_ref, theta_ref, concepts2_ref):
    dinv = dinv_ref[...][:, :64]
    h2 = (dinv * (s2_ref[0, :_N] + s2_ref[1, :_N] + hw2p_ref[...])
          + b2_ref[...][None, :])
    concepts = jax.nn.sigmoid(h2)
    pre = (jnp.dot(concepts, wt1c_ref[...], preferred_element_type=_f32)
           + pre_a_ref[...])
    theta = (jnp.dot(jnp.tanh(pre), wt2_ref[...], preferred_element_type=_f32)
             + bt2_ref[...][None, :])
    contrib = theta * concepts
    gids = lax.broadcasted_iota(jnp.int32, (_G, 1), 0)
    onehot_t = (jnp.reshape(batch_ref[...], (1, _N)) == gids).astype(_f32)
    pooled = jnp.dot(onehot_t, contrib,
                     preferred_element_type=_f32)
    logits = jnp.dot(pooled, wout_ref[...],
                     preferred_element_type=_f32) + bout_ref[...][None, :]
    m = jnp.max(logits, axis=-1, keepdims=True)
    lse = jnp.log(jnp.sum(jnp.exp(logits - m), axis=-1, keepdims=True))
    xout_ref[...] = logits - m - lse
    concepts_ref[...] = concepts
    theta_ref[...] = theta
    concepts2_ref[...] = concepts


def kernel(x, edge_index, batch, annotations, W1, b1, W2, b2, Wt1, bt1,
           Wt2, bt2, Wout, bout):
    dego = _deg_call(edge_index)

    hw1p, dinv = pl.pallas_call(
        _tc1_body,
        out_shape=(
            jax.ShapeDtypeStruct((_N, 128), _f32),
            jax.ShapeDtypeStruct((_N, 128), _f32),
        ),
    )(x, W1, dego)

    s1 = _seg_call(hw1p, edge_index, 128)

    hw2p = pl.pallas_call(
        _tc2_body,
        out_shape=jax.ShapeDtypeStruct((_N, 64), _f32),
    )(s1, hw1p, dinv, W2, b1)

    s2 = _seg_call(hw2p, edge_index, 64)

    pre_a = pl.pallas_call(
        _pre_a_body,
        out_shape=jax.ShapeDtypeStruct((_N, 64), _f32),
    )(annotations, Wt1[64:], bt1)

    x_out, concepts, theta, concepts2 = pl.pallas_call(
        _tc3_body,
        out_shape=(
            jax.ShapeDtypeStruct((_G, 10), _f32),
            jax.ShapeDtypeStruct((_N, 64), _f32),
            jax.ShapeDtypeStruct((_N, 64), _f32),
            jax.ShapeDtypeStruct((_N, 64), _f32),
        ),
    )(s2, hw2p, dinv, b2, pre_a, Wt1[:64], Wt2, bt2,
      Wout, bout, batch)

    return (x_out, concepts, theta, concepts2)


# speedup vs baseline: 35.5979x; 1.0052x over previous
import functools

import jax
import jax.numpy as jnp
from jax import lax
from jax.experimental import pallas as pl
from jax.experimental.pallas import tpu as pltpu
from jax.experimental.pallas import tpu_sc as plsc

_N = 10000
_E = 320000
_G = 64
_NC = 2
_NS = 16
_NW = _NC * _NS
_C = 80
_CH = (_E // _NW) // _C
_NB = 25
_NP = 10240
_RS = _NP // _NS

_f32 = jnp.float32


def _zero_fill(ref, rows, width):
    lanes = width // 16

    def body(k, _):
        i = k // lanes
        j = (k % lanes) * 16
        ref[i, pl.ds(j, 16)] = jnp.zeros((16,), _f32)
        return _

    lax.fori_loop(0, rows * lanes, body, None)


def _deg_body(edge, out, acc, dst_idx, ones_v, zbuf, sem):
    cid = lax.axis_index("c")
    sid = lax.axis_index("s")
    w = cid * _NS + sid

    def fill_ones(i, _):
        ones_v[i, :] = jnp.ones((16,), _f32)
        return _

    lax.fori_loop(0, _C, fill_ones, None)
    _zero_fill(zbuf, _RS, 16)
    pltpu.sync_copy(zbuf, acc.at[pl.ds(sid * _RS, _RS)])
    plsc.subcore_barrier()

    pltpu.sync_copy(edge.at[1, pl.ds(w * (_CH * _C), _CH * _C)], dst_idx)

    def super_step(s, _):
        base = s * _NB
        descs = []
        for b in range(_NB):
            idx = dst_idx.at[pl.ds(pl.multiple_of((base + b) * _C, _C), _C)]
            descs.append(
                pltpu.async_copy(ones_v, acc.at[idx], sem, add=True))
        for d in descs:
            d.wait()
        return _

    lax.fori_loop(0, _CH // _NB, super_step, None)
    plsc.subcore_barrier()
    pltpu.sync_copy(acc.at[pl.ds(sid * _RS, _RS)],
                    out.at[cid, pl.ds(sid * _RS, _RS)])


def _deg_call(edge_index):
    mesh = plsc.VectorSubcoreMesh(core_axis_name="c", subcore_axis_name="s")
    return pl.kernel(
        _deg_body,
        out_type=jax.ShapeDtypeStruct((_NC, _NP, 16), _f32),
        mesh=mesh,
        compiler_params=pltpu.CompilerParams(use_tc_tiling_on_sc=False),
        scratch_types=[
            pltpu.VMEM_SHARED((_NP, 16), _f32),
            pltpu.VMEM((_CH * _C,), jnp.int32),
            pltpu.VMEM((_C, 16), _f32),
            pltpu.VMEM((_RS, 16), _f32),
            pltpu.SemaphoreType.DMA,
        ],
    )(edge_index)


def _seg_body(table, edge, out, acc, src_idx, dst_idx, rows,
              gsem, ssem, *, width, nb, cw, chunks):
    cid = lax.axis_index("c")
    sid = lax.axis_index("s")
    w = cid * _NS + sid
    ew = cw * chunks

    _zero_fill(rows[0], cw, width)
    zd = []
    for t in range(_RS // cw):
        zd.append(pltpu.async_copy(
            rows[0], acc.at[pl.ds(sid * _RS + t * cw, cw)], gsem[0]))
    for d in zd:
        d.wait()
    plsc.subcore_barrier()

    pltpu.sync_copy(edge.at[0, pl.ds(w * ew, ew)], src_idx)
    pltpu.sync_copy(edge.at[1, pl.ds(w * ew, ew)], dst_idx)

    def chunk(ref, j):
        return ref.at[pl.ds(pl.multiple_of(j * cw, cw), cw)]

    def gather(j, b):
        return pltpu.make_async_copy(table.at[chunk(src_idx, j)], rows[b],
                                     gsem[b])

    def scatter(j, b):
        return pltpu.make_async_copy(rows[b], acc.at[chunk(dst_idx, j)],
                                     ssem[b])

    supers = chunks // nb
    for b in range(nb):
        gather(b, b).start()

    def super_step(s, _):
        base = s * nb
        for b in range(nb):
            gather(base + b, b).wait()
            scatter(base + b, b).start(add=True)

        @pl.when(s < supers - 1)
        def _refill():
            for b in range(nb):
                scatter(base + b, b).wait()
                gather(base + nb + b, b).start()

        return _

    lax.fori_loop(0, supers, super_step, None)
    for b in range(nb):
        scatter((supers - 1) * nb + b, b).wait()
    for j in range(supers * nb, chunks):
        gather(j, 0).start()
        gather(j, 0).wait()
        scatter(j, 0).start(add=True)
        scatter(j, 0).wait()
    plsc.subcore_barrier()
    pltpu.sync_copy(acc.at[pl.ds(sid * _RS, _RS)],
                    out.at[cid, pl.ds(sid * _RS, _RS)])


def _seg_call(table, edge_index, width):
    nb = 5 if width == 128 else 8
    cw = 40
    chunks = (_E // _NW) // cw
    mesh = plsc.VectorSubcoreMesh(core_axis_name="c", subcore_axis_name="s")
    return pl.kernel(
        functools.partial(_seg_body, width=width, nb=nb, cw=cw,
                          chunks=chunks),
        out_type=jax.ShapeDtypeStruct((_NC, _NP, width), _f32),
        mesh=mesh,
        compiler_params=pltpu.CompilerParams(use_tc_tiling_on_sc=False),
        scratch_types=[
            pltpu.VMEM_SHARED((_NP, width), _f32),
            pltpu.VMEM((cw * chunks,), jnp.int32),
            pltpu.VMEM((cw * chunks,), jnp.int32),
            [pltpu.VMEM((cw, width), _f32) for _ in range(nb)],
            [pltpu.SemaphoreType.DMA for _ in range(nb)],
            [pltpu.SemaphoreType.DMA for _ in range(nb)],
        ],
    )(table, edge_index)


def _dinv_from(dego_ref):
    deg = dego_ref[0, :_N] + dego_ref[1, :_N]
    d = deg[:, :1] + 1.0
    return lax.rsqrt(jnp.maximum(d, 1.0))


def _tc1_body(x_ref, w1_ref, dego_ref, hw1p_ref, dinv_ref):
    dinv = _dinv_from(dego_ref)
    dinv_ref[...] = jnp.broadcast_to(dinv, (_N, 128))
    hw1p_ref[...] = jnp.dot(x_ref[...], w1_ref[...],
                            preferred_element_type=_f32) * dinv


def _tc2_body(s1_ref, hw1p_ref, dinv_ref, w2_ref, b1_ref, hw2p_ref):
    dinv = dinv_ref[...]
    h1 = (dinv * (s1_ref[0, :_N] + s1_ref[1, :_N] + hw1p_ref[...])
          + b1_ref[...][None, :])
    h1 = jnp.maximum(h1, 0.0)
    hw2p_ref[...] = jnp.dot(h1, w2_ref[...],
                            preferred_element_type=_f32) * dinv[:, :64]


def _pre_a_body(ann_ref, wt1a_ref, bt1_ref, pre_a_ref):
    pre_a_ref[...] = (jnp.dot(ann_ref[...], wt1a_ref[...],
                              preferred_element_type=_f32)
                      + bt1_ref[...][None, :])


def _tc3_body(s2_ref, hw2p_ref, dinv_ref, b2_ref, pre_a_ref, wt1c_ref,
              wt2_ref, bt2_ref, wout_ref, bout_ref,
              batch_ref, xout_ref, concepts_ref, theta_ref, concepts2_ref):
    dinv = dinv_ref[...][:, :64]
    h2 = (dinv * (s2_ref[0, :_N] + s2_ref[1, :_N] + hw2p_ref[...])
          + b2_ref[...][None, :])
    concepts = jax.nn.sigmoid(h2)
    pre = (jnp.dot(concepts, wt1c_ref[...], preferred_element_type=_f32)
           + pre_a_ref[...])
    theta = (jnp.dot(jnp.tanh(pre), wt2_ref[...], preferred_element_type=_f32)
             + bt2_ref[...][None, :])
    contrib = theta * concepts
    gids = lax.broadcasted_iota(jnp.int32, (_G, 1), 0)
    onehot_t = (jnp.reshape(batch_ref[...], (1, _N)) == gids).astype(_f32)
    pooled = jnp.dot(onehot_t, contrib,
                     preferred_element_type=_f32)
    logits = jnp.dot(pooled, wout_ref[...],
                     preferred_element_type=_f32) + bout_ref[...][None, :]
    m = jnp.max(logits, axis=-1, keepdims=True)
    lse = jnp.log(jnp.sum(jnp.exp(logits - m), axis=-1, keepdims=True))
    xout_ref[...] = logits - m - lse
    concepts_ref[...] = concepts
    theta_ref[...] = theta
    concepts2_ref[...] = concepts


def kernel(x, edge_index, batch, annotations, W1, b1, W2, b2, Wt1, bt1,
           Wt2, bt2, Wout, bout):
    dego = _deg_call(edge_index)

    hw1p, dinv = pl.pallas_call(
        _tc1_body,
        out_shape=(
            jax.ShapeDtypeStruct((_N, 128), _f32),
            jax.ShapeDtypeStruct((_N, 128), _f32),
        ),
    )(x, W1, dego)

    s1 = _seg_call(hw1p, edge_index, 128)

    hw2p = pl.pallas_call(
        _tc2_body,
        out_shape=jax.ShapeDtypeStruct((_N, 64), _f32),
    )(s1, hw1p, dinv, W2, b1)

    s2 = _seg_call(hw2p, edge_index, 64)

    pre_a = pl.pallas_call(
        _pre_a_body,
        out_shape=jax.ShapeDtypeStruct((_N, 64), _f32),
    )(annotations, Wt1[64:], bt1)

    x_out, concepts, theta, concepts2 = pl.pallas_call(
        _tc3_body,
        out_shape=(
            jax.ShapeDtypeStruct((_G, 10), _f32),
            jax.ShapeDtypeStruct((_N, 64), _f32),
            jax.ShapeDtypeStruct((_N, 64), _f32),
            jax.ShapeDtypeStruct((_N, 64), _f32),
        ),
    )(s2, hw2p, dinv, b2, pre_a, Wt1[:64], Wt2, bt2,
      Wout, bout, batch)

    return (x_out, concepts, theta, concepts2)
